# Initial kernel scaffold; baseline (speedup 1.0000x reference)
import jax
import jax.numpy as jnp
from jax.experimental import pallas as pl


def kernel(features, nodes, neigh_idx, num_sample):
    raise NotImplementedError("write your pallas kernel here")


# speedup vs baseline: 1.8129x; 1.8129x over previous
import functools

import jax
import jax.numpy as jnp
from jax import lax
from jax.experimental import pallas as pl
from jax.experimental.pallas import tpu as pltpu
from jax.experimental.pallas import tpu_sc as plsc

_D = 128
_LANES = 16
_NW = 32
_C = 32


def _build(b_pad, s):
    b_per_w = b_pad // _NW
    n_chunks = b_per_w // _C
    rows = _C * s
    n_col = _D // _LANES

    mesh = plsc.VectorSubcoreMesh(core_axis_name="c", subcore_axis_name="s")

    @functools.partial(
        pl.kernel,
        mesh=mesh,
        out_type=jax.ShapeDtypeStruct((b_pad, _D), jnp.float32),
        scratch_types=[
            pltpu.VMEM((rows,), jnp.int32),
            pltpu.VMEM((rows,), jnp.int32),
            pltpu.VMEM((rows, _D), jnp.float32),
            pltpu.VMEM((rows, _D), jnp.float32),
            pltpu.VMEM((_C, _D), jnp.float32),
            pltpu.SemaphoreType.DMA,
            pltpu.SemaphoreType.DMA,
        ],
    )
    def aggregate(table_hbm, idx_hbm, out_hbm,
                  idx0, idx1, rows0, rows1, out_v, sem0, sem1):
        wid = lax.axis_index("s") * 2 + lax.axis_index("c")
        nbase = wid * b_per_w
        ibase = nbase * s
        idx_bufs = (idx0, idx1)
        row_bufs = (rows0, rows1)
        sems = (sem0, sem1)
        inv = jnp.float32(1.0 / s)

        def start(g, buf):
            pltpu.sync_copy(idx_hbm.at[pl.ds(ibase + g * rows, rows)],
                            idx_bufs[buf])
            pltpu.async_copy(table_hbm.at[idx_bufs[buf]], row_bufs[buf],
                             sems[buf])

        def wait(buf):
            pltpu.make_async_copy(table_hbm.at[idx_bufs[buf]], row_bufs[buf],
                                  sems[buf]).wait()

        def process(g, buf):
            wait(buf)

            @pl.when(g + 1 < n_chunks)
            def _():
                start(g + 1, 1 - buf)

            rbuf = row_bufs[buf]

            def node_body(c, carry):
                r0 = c * s
                for d in range(n_col):
                    sl = pl.ds(d * _LANES, _LANES)
                    acc = rbuf[r0, sl]
                    for j in range(1, s):
                        acc = acc + rbuf[r0 + j, sl]
                    out_v[c, sl] = acc * inv
                return carry

            lax.fori_loop(0, _C, node_body, 0)
            pltpu.sync_copy(out_v, out_hbm.at[pl.ds(nbase + g * _C, _C)])

        start(0, 0)

        def outer(i, carry):
            process(2 * i, 0)
            process(2 * i + 1, 1)
            return carry

        lax.fori_loop(0, n_chunks // 2, outer, 0)

    return aggregate


@functools.lru_cache(maxsize=None)
def _cached(b_pad, s):
    return _build(b_pad, s)


def kernel(features, nodes, neigh_idx, num_sample):
    del nodes
    b, s = neigh_idx.shape
    unit = _NW * _C * 2
    b_pad = ((b + unit - 1) // unit) * unit
    idx_flat = jnp.pad(neigh_idx, ((0, b_pad - b), (0, 0))).reshape(-1)
    out = _cached(b_pad, s)(features, idx_flat)
    return out[:b]


# speedup vs baseline: 1.8707x; 1.0318x over previous
import functools

import jax
import jax.numpy as jnp
from jax import lax
from jax.experimental import pallas as pl
from jax.experimental.pallas import tpu as pltpu
from jax.experimental.pallas import tpu_sc as plsc

_D = 128
_LANES = 16
_NW = 32
_C = 40


def _tree_sum(vals):
    while len(vals) > 1:
        nxt = [a + b for a, b in zip(vals[0::2], vals[1::2])]
        if len(vals) % 2:
            nxt.append(vals[-1])
        vals = nxt
    return vals[0]


def _build(b_pad, s):
    b_per_w = b_pad // _NW
    n_chunks = b_per_w // _C
    rows = _C * s
    n_col = _D // _LANES

    mesh = plsc.VectorSubcoreMesh(core_axis_name="c", subcore_axis_name="s")

    @functools.partial(
        pl.kernel,
        mesh=mesh,
        out_type=jax.ShapeDtypeStruct((b_pad, _D), jnp.float32),
        scratch_types=[
            pltpu.VMEM((b_per_w * s,), jnp.int32),
            pltpu.VMEM((rows, _D), jnp.float32),
            pltpu.VMEM((rows, _D), jnp.float32),
            pltpu.VMEM((_C, _D), jnp.float32),
            pltpu.SemaphoreType.DMA,
            pltpu.SemaphoreType.DMA,
        ],
    )
    def aggregate(table_hbm, idx_hbm, out_hbm,
                  idx_all, rows0, rows1, out_v, sem0, sem1):
        wid = lax.axis_index("s") * 2 + lax.axis_index("c")
        nbase = wid * b_per_w
        ibase = nbase * s
        row_bufs = (rows0, rows1)
        sems = (sem0, sem1)
        inv = jnp.float32(1.0 / s)

        pltpu.sync_copy(idx_hbm.at[pl.ds(ibase, b_per_w * s)], idx_all)

        def start(g, buf):
            pltpu.async_copy(table_hbm.at[idx_all.at[pl.ds(g * rows, rows)]],
                             row_bufs[buf], sems[buf])

        def wait(g, buf):
            pltpu.make_async_copy(
                table_hbm.at[idx_all.at[pl.ds(g * rows, rows)]],
                row_bufs[buf], sems[buf]).wait()

        def process(g, buf):
            wait(g, buf)

            @pl.when(g + 1 < n_chunks)
            def _():
                start(g + 1, 1 - buf)

            rbuf = row_bufs[buf]

            def node_body(c, carry):
                r0 = c * s
                for d in range(n_col):
                    sl = pl.ds(d * _LANES, _LANES)
                    acc = _tree_sum([rbuf[r0 + j, sl] for j in range(s)])
                    out_v[c, sl] = acc * inv
                return carry

            lax.fori_loop(0, _C, node_body, 0)
            pltpu.sync_copy(out_v, out_hbm.at[pl.ds(nbase + g * _C, _C)])

        start(0, 0)

        def outer(i, carry):
            process(2 * i, 0)
            process(2 * i + 1, 1)
            return carry

        lax.fori_loop(0, n_chunks // 2, outer, 0)

    return aggregate


@functools.lru_cache(maxsize=None)
def _cached(b_pad, s):
    return _build(b_pad, s)


def kernel(features, nodes, neigh_idx, num_sample):
    del nodes
    b, s = neigh_idx.shape
    unit = _NW * _C * 2
    b_pad = ((b + unit - 1) // unit) * unit
    idx_flat = jnp.pad(neigh_idx, ((0, b_pad - b), (0, 0))).reshape(-1)
    out = _cached(b_pad, s)(features, idx_flat)
    return out[:b]


# speedup vs baseline: 4.0598x; 2.1703x over previous
import functools

import jax
import jax.numpy as jnp
from jax import lax
from jax.experimental import pallas as pl
from jax.experimental.pallas import tpu as pltpu
from jax.experimental.pallas import tpu_sc as plsc

_D = 128
_LANES = 16
_NSC = 16
_C = 32
_HW = 64
_PW = _HW // 2


def _build(b_pad, v_pad, s):
    nt = b_pad // _NSC
    n_chunks = nt // _C
    rows = _C * s
    vt = v_pad // _NSC
    c2 = _C // 2
    b2 = b_pad // 2

    mesh = plsc.VectorSubcoreMesh(core_axis_name="c", subcore_axis_name="s")

    @functools.partial(
        pl.kernel,
        mesh=mesh,
        compiler_params=pltpu.CompilerParams(needs_layout_passes=False, use_tc_tiling_on_sc=False),
        out_type=jax.ShapeDtypeStruct((2, b2, _D), jnp.float32),
        scratch_types=[
            pltpu.VMEM_SHARED((v_pad, _PW), jnp.int32),
            pltpu.VMEM((rows,), jnp.int32),
            pltpu.VMEM((rows,), jnp.int32),
            pltpu.VMEM((rows, _PW), jnp.int32),
            pltpu.VMEM((rows, _PW), jnp.int32),
            pltpu.VMEM((c2, _D), jnp.float32),
            pltpu.VMEM((c2, _D), jnp.float32),
            pltpu.SemaphoreType.DMA,
            pltpu.SemaphoreType.DMA,
            pltpu.SemaphoreType.DMA,
            pltpu.SemaphoreType.DMA,
            pltpu.SemaphoreType.DMA,
            pltpu.SemaphoreType.DMA,
        ],
    )
    def aggregate(tq_hbm, idx_hbm, out_hbm,
                  spmem, idx0, idx1, rows0, rows1, outv0, outv1,
                  isem0, isem1, gsem0, gsem1, osem0, osem1):
        cid = lax.axis_index("c")
        sid = lax.axis_index("s")
        nbase = sid * nt
        ibase = nbase * s
        pbase = pl.multiple_of(nbase // 2, 8)
        idx_bufs = (idx0, idx1)
        row_bufs = (rows0, rows1)
        out_bufs = (outv0, outv1)
        isems = (isem0, isem1)
        gsems = (gsem0, gsem1)
        osems = (osem0, osem1)
        inv = jnp.float32(1.0 / s)

        vb = pl.multiple_of(sid * vt, 16)
        pltpu.sync_copy(tq_hbm.at[cid, pl.ds(vb, vt)],
                        spmem.at[pl.ds(vb, vt)])
        plsc.subcore_barrier()

        def prefetch(g, buf):
            pltpu.async_copy(idx_hbm.at[pl.ds(ibase + g * rows, rows)],
                             idx_bufs[buf], isems[buf])

        def fire(g, buf):
            pltpu.make_async_copy(
                idx_hbm.at[pl.ds(ibase + g * rows, rows)],
                idx_bufs[buf], isems[buf]).wait()
            pltpu.async_copy(spmem.at[idx_bufs[buf]], row_bufs[buf],
                             gsems[buf])

        def wait_rows(buf):
            pltpu.make_async_copy(spmem.at[idx_bufs[buf]], row_bufs[buf],
                                  gsems[buf]).wait()

        def process(g, buf):
            wait_rows(buf)

            @pl.when(g + 2 < n_chunks)
            def _():
                prefetch(g + 2, buf)

            @pl.when(g + 1 < n_chunks)
            def _():
                fire(g + 1, 1 - buf)

            rbuf = row_bufs[buf]
            obuf = out_bufs[buf]

            @pl.when(g >= 2)
            def _():
                pltpu.make_async_copy(
                    obuf, out_hbm.at[0, pl.ds(pbase, c2)], osems[buf]).wait()

            def node_body(c, carry):
                r0 = c * s
                orow = c // 2
                obase = (c % 2) * _HW
                for g16 in range(2):
                    sl = pl.ds(g16 * _LANES, _LANES)
                    acc_lo = None
                    acc_hi = None
                    for j in range(s):
                        x = plsc.bitcast(rbuf[r0 + j, sl], jnp.bfloat16)
                        lo, hi = plsc.unpack(x, format=plsc.PackFormat.INTERLEAVED)
                        acc_lo = lo if acc_lo is None else acc_lo + lo
                        acc_hi = hi if acc_hi is None else acc_hi + hi
                    obuf[orow, pl.ds(obase + 32 * g16, _LANES)] = acc_lo * inv
                    obuf[orow, pl.ds(obase + 32 * g16 + _LANES, _LANES)] = (
                        acc_hi * inv)
                return carry

            lax.fori_loop(0, _C, node_body, 0)
            ob = pl.multiple_of(pbase + g * c2, 8)
            pltpu.async_copy(obuf, out_hbm.at[cid, pl.ds(ob, c2)],
                             osems[buf])

        prefetch(0, 0)
        prefetch(1, 1)
        fire(0, 0)

        def outer(i, carry):
            process(2 * i, 0)
            process(2 * i + 1, 1)
            return carry

        lax.fori_loop(0, n_chunks // 2, outer, 0)

        pltpu.make_async_copy(outv0, out_hbm.at[0, pl.ds(pbase, c2)],
                              osem0).wait()
        pltpu.make_async_copy(outv1, out_hbm.at[0, pl.ds(pbase, c2)],
                              osem1).wait()

    return aggregate


@functools.lru_cache(maxsize=None)
def _cached(b_pad, v_pad, s):
    return _build(b_pad, v_pad, s)


def _pack_halves(features, v_pad):
    v, d = features.shape
    fpad = jnp.pad(features, ((0, v_pad - v), (0, 0)))
    hb = fpad.reshape(v_pad, 2, 2, 2, 16).astype(jnp.bfloat16)
    arr = hb.transpose(1, 0, 2, 4, 3)
    packed = lax.bitcast_convert_type(
        arr.reshape(2, v_pad, _PW, 2), jnp.int32)
    return packed


def kernel(features, nodes, neigh_idx, num_sample):
    del nodes
    b, s = neigh_idx.shape
    v, d = features.shape
    unit = _NSC * _C * 2
    b_pad = ((b + unit - 1) // unit) * unit
    v_pad = ((v + _NSC * 16 - 1) // (_NSC * 16)) * (_NSC * 16)
    idx_flat = jnp.pad(neigh_idx, ((0, b_pad - b), (0, 0))).reshape(-1)
    tq = _pack_halves(features, v_pad)
    out_p = _cached(b_pad, v_pad, s)(tq, idx_flat)
    out = (out_p.reshape(2, b_pad // 2, 2, _HW)
           .transpose(1, 2, 0, 3)
           .reshape(b_pad, d))
    return out[:b]
